# baseline (speedup 1.0000x reference)
import numpy as np
import jax
import jax.numpy as jnp
from jax.experimental import pallas as pl
from jax.experimental.pallas import tpu as pltpu

_B, _T, _D, _F, _E, _K, _C = 1, 2048, 1024, 2048, 4, 2, 2000
_FC = 256
_NFC = _F // _FC
_TT = 256
_NTT = _T // _TT
_EPAD = 128


def _pos_encoding(t, d):
    pos = np.arange(t)[:, None].astype(np.float32)
    i = np.arange(0, d, 2)[None, :].astype(np.float32)
    angle = pos / np.power(10000.0, i / d)
    pe = np.zeros((t, d), dtype=np.float32)
    pe[:, 0::2] = np.sin(angle)
    pe[:, 1::2] = np.cos(angle)
    return pe


_PE = _pos_encoding(_T, _D)


def _stage1_kernel(x_ref, fc1w_ref, fc1b_ref, g_ref, b_ref, mu_ref, var_ref,
                   pe_ref, rw_ref, rb_ref, h_ref, rlog_ref):
    h = jnp.dot(x_ref[...], fc1w_ref[...], preferred_element_type=jnp.float32)
    h = h + fc1b_ref[...]
    scale = g_ref[...] * jax.lax.rsqrt(var_ref[...] + 1e-5)
    shift = b_ref[...] - mu_ref[...] * scale
    h = h * scale + shift
    h = jnp.maximum(h, 0.0) + pe_ref[...]
    h_ref[...] = h
    rlog_ref[...] = jnp.dot(h, rw_ref[...], preferred_element_type=jnp.float32) + rb_ref[...]


def _stage1(x2, fc1_w, fc1_b, bn_gamma, bn_beta, bn_mean, bn_var, router_w, router_b):
    rw_p = jnp.zeros((_D, _EPAD), jnp.float32).at[:, :_E].set(router_w)
    rb_p = jnp.zeros((1, _EPAD), jnp.float32).at[0, :_E].set(router_b)
    row = lambda v: v.reshape(1, _D)
    pe = jnp.asarray(_PE)
    h, rlog = pl.pallas_call(
        _stage1_kernel,
        grid=(_NTT,),
        in_specs=[
            pl.BlockSpec((_TT, _D), lambda i: (i, 0)),
            pl.BlockSpec((_D, _D), lambda i: (0, 0)),
            pl.BlockSpec((1, _D), lambda i: (0, 0)),
            pl.BlockSpec((1, _D), lambda i: (0, 0)),
            pl.BlockSpec((1, _D), lambda i: (0, 0)),
            pl.BlockSpec((1, _D), lambda i: (0, 0)),
            pl.BlockSpec((1, _D), lambda i: (0, 0)),
            pl.BlockSpec((_TT, _D), lambda i: (i, 0)),
            pl.BlockSpec((_D, _EPAD), lambda i: (0, 0)),
            pl.BlockSpec((1, _EPAD), lambda i: (0, 0)),
        ],
        out_specs=[
            pl.BlockSpec((_TT, _D), lambda i: (i, 0)),
            pl.BlockSpec((_TT, _EPAD), lambda i: (i, 0)),
        ],
        out_shape=[
            jax.ShapeDtypeStruct((_T, _D), jnp.float32),
            jax.ShapeDtypeStruct((_T, _EPAD), jnp.float32),
        ],
    )(x2, fc1_w, row(fc1_b), row(bn_gamma), row(bn_beta), row(bn_mean),
      row(bn_var), pe, rw_p, rb_p)
    return h, rlog


def _route_kernel(rlog_ref, comb_ref, aux_ref):
    cols = [rlog_ref[:, i:i + 1] for i in range(_E)]
    m = jnp.maximum(jnp.maximum(cols[0], cols[1]), jnp.maximum(cols[2], cols[3]))
    es = [jnp.exp(c - m) for c in cols]
    ssum = es[0] + es[1] + es[2] + es[3]
    gates = [e / ssum for e in es]
    m1 = jnp.maximum(jnp.maximum(gates[0], gates[1]), jnp.maximum(gates[2], gates[3]))
    i1 = jnp.where(gates[0] == m1, 0,
         jnp.where(gates[1] == m1, 1,
         jnp.where(gates[2] == m1, 2, 3))).astype(jnp.int32)
    g2 = [jnp.where(i1 == i, -1.0, gates[i]) for i in range(_E)]
    m2 = jnp.maximum(jnp.maximum(g2[0], g2[1]), jnp.maximum(g2[2], g2[3]))
    i2 = jnp.where(g2[0] == m2, 0,
         jnp.where(g2[1] == m2, 1,
         jnp.where(g2[2] == m2, 2, 3))).astype(jnp.int32)
    denom = m1 + m2
    aux_acc = None
    for i in range(_E):
        sel = jnp.logical_or(i1 == i, i2 == i)
        ci = jnp.where(sel, gates[i] / denom, 0.0)
        comb_ref[:, i:i + 1] = ci
        imp = jnp.sum(gates[i], axis=0, keepdims=True) / _T
        load = jnp.sum((ci > 0).astype(jnp.float32), axis=0, keepdims=True) / _T
        term = imp * load
        aux_acc = term if aux_acc is None else aux_acc + term
    aux_ref[...] = _E * aux_acc


def _stage2(rlog):
    comb, aux = pl.pallas_call(
        _route_kernel,
        in_specs=[pl.BlockSpec((_T, _EPAD), lambda: (0, 0))],
        out_specs=[
            pl.BlockSpec((_T, _E), lambda: (0, 0)),
            pl.BlockSpec((1, 1), lambda: (0, 0)),
        ],
        out_shape=[
            jax.ShapeDtypeStruct((_T, _E), jnp.float32),
            jax.ShapeDtypeStruct((1, 1), jnp.float32),
        ],
    )(rlog)
    return comb, aux


def _moe_kernel(hpad_ref, w1_ref, b1_ref, w2_ref, b2_ref, comb_ref, out_ref, y1_ref):
    e = pl.program_id(0)
    fc = pl.program_id(1)

    @pl.when(jnp.logical_and(e == 0, fc == 0))
    def _init():
        out_ref[...] = jnp.zeros_like(out_ref)

    hp = hpad_ref[...]
    w1 = w1_ref[0].astype(jnp.bfloat16)
    b1 = b1_ref[0]
    w2 = w2_ref[0].astype(jnp.bfloat16)
    b2 = b2_ref[0]
    c = comb_ref[0]

    y1 = jnp.dot(hp[1:_T + 1], w1[0], preferred_element_type=jnp.float32)
    y1 += jnp.dot(hp[2:_T + 2], w1[1], preferred_element_type=jnp.float32)
    y1 += jnp.dot(hp[3:_T + 3], w1[2], preferred_element_type=jnp.float32)
    y1 = jnp.maximum(y1 + b1, 0.0).astype(jnp.bfloat16)
    y1_ref[0:1, :] = jnp.zeros((1, _FC), jnp.bfloat16)
    y1_ref[1:_T + 1, :] = y1
    y1_ref[_T + 1:_T + 2, :] = jnp.zeros((1, _FC), jnp.bfloat16)

    y = jnp.dot(y1_ref[0:_T], w2[0], preferred_element_type=jnp.float32)
    y += jnp.dot(y1_ref[1:_T + 1], w2[1], preferred_element_type=jnp.float32)
    y += jnp.dot(y1_ref[2:_T + 2], w2[2], preferred_element_type=jnp.float32)

    @pl.when(fc == 0)
    def _bias():
        out_ref[...] += c * b2

    out_ref[...] += c * y


def _stage3(h, exp_w1, exp_b1, exp_w2, exp_b2, comb):
    hpad = jnp.pad(h, ((2, 6), (0, 0))).astype(jnp.bfloat16)
    comb_t = comb.T.reshape(_E, _T, 1)
    b1r = exp_b1.reshape(_E, 1, _F)
    b2r = exp_b2.reshape(_E, 1, _D)
    moe = pl.pallas_call(
        _moe_kernel,
        grid=(_E, _NFC),
        in_specs=[
            pl.BlockSpec((_T + 8, _D), lambda e, fc: (0, 0)),
            pl.BlockSpec((1, 3, _D, _FC), lambda e, fc: (e, 0, 0, fc)),
            pl.BlockSpec((1, 1, _FC), lambda e, fc: (e, 0, fc)),
            pl.BlockSpec((1, 3, _FC, _D), lambda e, fc: (e, 0, fc, 0)),
            pl.BlockSpec((1, 1, _D), lambda e, fc: (e, 0, 0)),
            pl.BlockSpec((1, _T, 1), lambda e, fc: (e, 0, 0)),
        ],
        out_specs=pl.BlockSpec((_T, _D), lambda e, fc: (0, 0)),
        out_shape=jax.ShapeDtypeStruct((_T, _D), jnp.float32),
        scratch_shapes=[pltpu.VMEM((_T + 8, _FC), jnp.bfloat16)],
    )(hpad, exp_w1, b1r, exp_w2, b2r, comb_t)
    return moe


def _head_kernel(moe_ref, lng_ref, lnb_ref, ow_ref, ob_ref,
                 feat_ref, featn_ref, logits_ref, logp_ref, p_ref):
    m = moe_ref[...]
    mu = jnp.mean(m, axis=-1, keepdims=True)
    d = m - mu
    var = jnp.mean(d * d, axis=-1, keepdims=True)
    feat = d * jax.lax.rsqrt(var + 1e-6) * lng_ref[...] + lnb_ref[...]
    feat_ref[...] = feat
    n = jnp.sqrt(jnp.sum(feat * feat, axis=-1, keepdims=True))
    featn_ref[...] = feat / jnp.maximum(n, 1e-12)
    logits = jnp.dot(feat.astype(jnp.bfloat16), ow_ref[...].astype(jnp.bfloat16),
                     preferred_element_type=jnp.float32) + ob_ref[...]
    logits_ref[...] = logits
    mx = jnp.max(logits, axis=-1, keepdims=True)
    ex = jnp.exp(logits - mx)
    s = jnp.sum(ex, axis=-1, keepdims=True)
    logp_ref[...] = logits - (mx + jnp.log(s))
    p_ref[...] = ex / s


def _stage4(moe, ln_gamma, ln_beta, out_w, out_b):
    row = lambda v, n: v.reshape(1, n)
    outs = pl.pallas_call(
        _head_kernel,
        grid=(_NTT,),
        in_specs=[
            pl.BlockSpec((_TT, _D), lambda i: (i, 0)),
            pl.BlockSpec((1, _D), lambda i: (0, 0)),
            pl.BlockSpec((1, _D), lambda i: (0, 0)),
            pl.BlockSpec((_D, _C), lambda i: (0, 0)),
            pl.BlockSpec((1, _C), lambda i: (0, 0)),
        ],
        out_specs=[
            pl.BlockSpec((_TT, _D), lambda i: (i, 0)),
            pl.BlockSpec((_TT, _D), lambda i: (i, 0)),
            pl.BlockSpec((_TT, _C), lambda i: (i, 0)),
            pl.BlockSpec((_TT, _C), lambda i: (i, 0)),
            pl.BlockSpec((_TT, _C), lambda i: (i, 0)),
        ],
        out_shape=[
            jax.ShapeDtypeStruct((_T, _D), jnp.float32),
            jax.ShapeDtypeStruct((_T, _D), jnp.float32),
            jax.ShapeDtypeStruct((_T, _C), jnp.float32),
            jax.ShapeDtypeStruct((_T, _C), jnp.float32),
            jax.ShapeDtypeStruct((_T, _C), jnp.float32),
        ],
    )(moe, row(ln_gamma, _D), row(ln_beta, _D), out_w, row(out_b, _C))
    return outs


def kernel(x, mask, fc1_w, fc1_b, bn_gamma, bn_beta, bn_mean, bn_var,
           router_w, router_b, exp_w1, exp_b1, exp_w2, exp_b2,
           ln_gamma, ln_beta, out_w, out_b):
    x2 = x.reshape(_T, _D)
    h, rlog = _stage1(x2, fc1_w, fc1_b, bn_gamma, bn_beta, bn_mean, bn_var,
                      router_w, router_b)
    comb, aux = _stage2(rlog)
    moe = _stage3(h, exp_w1, exp_b1, exp_w2, exp_b2, comb)
    feat, featn, logits, logp, p = _stage4(moe, ln_gamma, ln_beta, out_w, out_b)
    r3 = lambda a: a.reshape(_B, _T, a.shape[-1])
    return (r3(feat), r3(featn), r3(logits), r3(logp), r3(p), aux[0, 0])


# speedup vs baseline: 1.4966x; 1.4966x over previous
import numpy as np
import jax
import jax.numpy as jnp
from jax.experimental import pallas as pl
from jax.experimental.pallas import tpu as pltpu

_B, _T, _D, _F, _E, _K, _C = 1, 2048, 1024, 2048, 4, 2, 2000
_FC = 512
_NFC = _F // _FC
_TT = 256
_NTT = _T // _TT
_EPAD = 128


def _pos_encoding(t, d):
    pos = np.arange(t)[:, None].astype(np.float32)
    i = np.arange(0, d, 2)[None, :].astype(np.float32)
    angle = pos / np.power(10000.0, i / d)
    pe = np.zeros((t, d), dtype=np.float32)
    pe[:, 0::2] = np.sin(angle)
    pe[:, 1::2] = np.cos(angle)
    return pe


_PE = _pos_encoding(_T, _D)


def _stage1_kernel(x_ref, fc1w_ref, fc1b_ref, g_ref, b_ref, mu_ref, var_ref,
                   pe_ref, rw_ref, rb_ref, h_ref, rlog_ref):
    h = jnp.dot(x_ref[...], fc1w_ref[...], preferred_element_type=jnp.float32)
    h = h + fc1b_ref[...]
    scale = g_ref[...] * jax.lax.rsqrt(var_ref[...] + 1e-5)
    shift = b_ref[...] - mu_ref[...] * scale
    h = h * scale + shift
    h = jnp.maximum(h, 0.0) + pe_ref[...]
    h_ref[...] = h
    rlog_ref[...] = jnp.dot(h, rw_ref[...], preferred_element_type=jnp.float32) + rb_ref[...]


def _stage1(x2, fc1_w, fc1_b, bn_gamma, bn_beta, bn_mean, bn_var, router_w, router_b):
    rw_p = jnp.zeros((_D, _EPAD), jnp.float32).at[:, :_E].set(router_w)
    rb_p = jnp.zeros((1, _EPAD), jnp.float32).at[0, :_E].set(router_b)
    row = lambda v: v.reshape(1, _D)
    pe = jnp.asarray(_PE)
    h, rlog = pl.pallas_call(
        _stage1_kernel,
        grid=(_NTT,),
        in_specs=[
            pl.BlockSpec((_TT, _D), lambda i: (i, 0)),
            pl.BlockSpec((_D, _D), lambda i: (0, 0)),
            pl.BlockSpec((1, _D), lambda i: (0, 0)),
            pl.BlockSpec((1, _D), lambda i: (0, 0)),
            pl.BlockSpec((1, _D), lambda i: (0, 0)),
            pl.BlockSpec((1, _D), lambda i: (0, 0)),
            pl.BlockSpec((1, _D), lambda i: (0, 0)),
            pl.BlockSpec((_TT, _D), lambda i: (i, 0)),
            pl.BlockSpec((_D, _EPAD), lambda i: (0, 0)),
            pl.BlockSpec((1, _EPAD), lambda i: (0, 0)),
        ],
        out_specs=[
            pl.BlockSpec((_TT, _D), lambda i: (i, 0)),
            pl.BlockSpec((_TT, _EPAD), lambda i: (i, 0)),
        ],
        out_shape=[
            jax.ShapeDtypeStruct((_T, _D), jnp.float32),
            jax.ShapeDtypeStruct((_T, _EPAD), jnp.float32),
        ],
    )(x2, fc1_w, row(fc1_b), row(bn_gamma), row(bn_beta), row(bn_mean),
      row(bn_var), pe, rw_p, rb_p)
    return h, rlog


def _route_kernel(rlog_ref, comb_ref, aux_ref):
    cols = [rlog_ref[:, i:i + 1] for i in range(_E)]
    m = jnp.maximum(jnp.maximum(cols[0], cols[1]), jnp.maximum(cols[2], cols[3]))
    es = [jnp.exp(c - m) for c in cols]
    ssum = es[0] + es[1] + es[2] + es[3]
    gates = [e / ssum for e in es]
    m1 = jnp.maximum(jnp.maximum(gates[0], gates[1]), jnp.maximum(gates[2], gates[3]))
    i1 = jnp.where(gates[0] == m1, 0,
         jnp.where(gates[1] == m1, 1,
         jnp.where(gates[2] == m1, 2, 3))).astype(jnp.int32)
    g2 = [jnp.where(i1 == i, -1.0, gates[i]) for i in range(_E)]
    m2 = jnp.maximum(jnp.maximum(g2[0], g2[1]), jnp.maximum(g2[2], g2[3]))
    i2 = jnp.where(g2[0] == m2, 0,
         jnp.where(g2[1] == m2, 1,
         jnp.where(g2[2] == m2, 2, 3))).astype(jnp.int32)
    denom = m1 + m2
    aux_acc = None
    for i in range(_E):
        sel = jnp.logical_or(i1 == i, i2 == i)
        ci = jnp.where(sel, gates[i] / denom, 0.0)
        comb_ref[:, i:i + 1] = ci
        imp = jnp.sum(gates[i], axis=0, keepdims=True) / _T
        load = jnp.sum((ci > 0).astype(jnp.float32), axis=0, keepdims=True) / _T
        term = imp * load
        aux_acc = term if aux_acc is None else aux_acc + term
    aux_ref[...] = _E * aux_acc


def _stage2(rlog):
    comb, aux = pl.pallas_call(
        _route_kernel,
        in_specs=[pl.BlockSpec((_T, _EPAD), lambda: (0, 0))],
        out_specs=[
            pl.BlockSpec((_T, _E), lambda: (0, 0)),
            pl.BlockSpec((1, 1), lambda: (0, 0)),
        ],
        out_shape=[
            jax.ShapeDtypeStruct((_T, _E), jnp.float32),
            jax.ShapeDtypeStruct((1, 1), jnp.float32),
        ],
    )(rlog)
    return comb, aux


def _moe_kernel(hpad_ref, w1_ref, b1_ref, w2_ref, b2_ref, comb_ref, out_ref, y1_ref):
    e = pl.program_id(0)
    fc = pl.program_id(1)

    @pl.when(jnp.logical_and(e == 0, fc == 0))
    def _init():
        out_ref[...] = jnp.zeros_like(out_ref)

    hp = hpad_ref[...]
    w1 = w1_ref[0].astype(jnp.bfloat16)
    b1 = b1_ref[0]
    w2 = w2_ref[0].astype(jnp.bfloat16)
    b2 = b2_ref[0]
    c = comb_ref[0]

    y1 = jnp.dot(hp[1:_T + 1], w1[0], preferred_element_type=jnp.float32)
    y1 += jnp.dot(hp[2:_T + 2], w1[1], preferred_element_type=jnp.float32)
    y1 += jnp.dot(hp[3:_T + 3], w1[2], preferred_element_type=jnp.float32)
    y1 = jnp.maximum(y1 + b1, 0.0).astype(jnp.bfloat16)
    y1_ref[0:1, :] = jnp.zeros((1, _FC), jnp.bfloat16)
    y1_ref[1:_T + 1, :] = y1
    y1_ref[_T + 1:_T + 2, :] = jnp.zeros((1, _FC), jnp.bfloat16)

    y = jnp.dot(y1_ref[0:_T], w2[0], preferred_element_type=jnp.float32)
    y += jnp.dot(y1_ref[1:_T + 1], w2[1], preferred_element_type=jnp.float32)
    y += jnp.dot(y1_ref[2:_T + 2], w2[2], preferred_element_type=jnp.float32)

    @pl.when(fc == 0)
    def _bias():
        out_ref[...] += c * b2

    out_ref[...] += c * y


def _stage3(h, exp_w1, exp_b1, exp_w2, exp_b2, comb):
    hpad = jnp.pad(h, ((2, 6), (0, 0))).astype(jnp.bfloat16)
    comb_t = comb.T.reshape(_E, _T, 1)
    b1r = exp_b1.reshape(_E, 1, _F)
    b2r = exp_b2.reshape(_E, 1, _D)
    moe = pl.pallas_call(
        _moe_kernel,
        grid=(_E, _NFC),
        in_specs=[
            pl.BlockSpec((_T + 8, _D), lambda e, fc: (0, 0)),
            pl.BlockSpec((1, 3, _D, _FC), lambda e, fc: (e, 0, 0, fc)),
            pl.BlockSpec((1, 1, _FC), lambda e, fc: (e, 0, fc)),
            pl.BlockSpec((1, 3, _FC, _D), lambda e, fc: (e, 0, fc, 0)),
            pl.BlockSpec((1, 1, _D), lambda e, fc: (e, 0, 0)),
            pl.BlockSpec((1, _T, 1), lambda e, fc: (e, 0, 0)),
        ],
        out_specs=pl.BlockSpec((_T, _D), lambda e, fc: (0, 0)),
        out_shape=jax.ShapeDtypeStruct((_T, _D), jnp.float32),
        scratch_shapes=[pltpu.VMEM((_T + 8, _FC), jnp.bfloat16)],
    )(hpad, exp_w1, b1r, exp_w2, b2r, comb_t)
    return moe


def _head_kernel(moe_ref, lng_ref, lnb_ref, ow_ref, ob_ref,
                 feat_ref, featn_ref, logits_ref, logp_ref, p_ref):
    m = moe_ref[...]
    mu = jnp.mean(m, axis=-1, keepdims=True)
    d = m - mu
    var = jnp.mean(d * d, axis=-1, keepdims=True)
    feat = d * jax.lax.rsqrt(var + 1e-6) * lng_ref[...] + lnb_ref[...]
    feat_ref[...] = feat
    n = jnp.sqrt(jnp.sum(feat * feat, axis=-1, keepdims=True))
    featn_ref[...] = feat / jnp.maximum(n, 1e-12)
    logits = jnp.dot(feat.astype(jnp.bfloat16), ow_ref[...].astype(jnp.bfloat16),
                     preferred_element_type=jnp.float32) + ob_ref[...]
    logits_ref[...] = logits
    mx = jnp.max(logits, axis=-1, keepdims=True)
    ex = jnp.exp(logits - mx)
    s = jnp.sum(ex, axis=-1, keepdims=True)
    logp_ref[...] = logits - (mx + jnp.log(s))
    p_ref[...] = ex / s


def _stage4(moe, ln_gamma, ln_beta, out_w, out_b):
    row = lambda v, n: v.reshape(1, n)
    outs = pl.pallas_call(
        _head_kernel,
        grid=(_NTT,),
        in_specs=[
            pl.BlockSpec((_TT, _D), lambda i: (i, 0)),
            pl.BlockSpec((1, _D), lambda i: (0, 0)),
            pl.BlockSpec((1, _D), lambda i: (0, 0)),
            pl.BlockSpec((_D, _C), lambda i: (0, 0)),
            pl.BlockSpec((1, _C), lambda i: (0, 0)),
        ],
        out_specs=[
            pl.BlockSpec((_TT, _D), lambda i: (i, 0)),
            pl.BlockSpec((_TT, _D), lambda i: (i, 0)),
            pl.BlockSpec((_TT, _C), lambda i: (i, 0)),
            pl.BlockSpec((_TT, _C), lambda i: (i, 0)),
            pl.BlockSpec((_TT, _C), lambda i: (i, 0)),
        ],
        out_shape=[
            jax.ShapeDtypeStruct((_T, _D), jnp.float32),
            jax.ShapeDtypeStruct((_T, _D), jnp.float32),
            jax.ShapeDtypeStruct((_T, _C), jnp.float32),
            jax.ShapeDtypeStruct((_T, _C), jnp.float32),
            jax.ShapeDtypeStruct((_T, _C), jnp.float32),
        ],
    )(moe, row(ln_gamma, _D), row(ln_beta, _D), out_w, row(out_b, _C))
    return outs


def kernel(x, mask, fc1_w, fc1_b, bn_gamma, bn_beta, bn_mean, bn_var,
           router_w, router_b, exp_w1, exp_b1, exp_w2, exp_b2,
           ln_gamma, ln_beta, out_w, out_b):
    x2 = x.reshape(_T, _D)
    h, rlog = _stage1(x2, fc1_w, fc1_b, bn_gamma, bn_beta, bn_mean, bn_var,
                      router_w, router_b)
    comb, aux = _stage2(rlog)
    moe = _stage3(h, exp_w1, exp_b1, exp_w2, exp_b2, comb)
    feat, featn, logits, logp, p = _stage4(moe, ln_gamma, ln_beta, out_w, out_b)
    r3 = lambda a: a.reshape(_B, _T, a.shape[-1])
    return (r3(feat), r3(featn), r3(logits), r3(logp), r3(p), aux[0, 0])


# speedup vs baseline: 1.5435x; 1.0313x over previous
import numpy as np
import jax
import jax.numpy as jnp
from jax.experimental import pallas as pl
from jax.experimental.pallas import tpu as pltpu

_B, _T, _D, _F, _E, _K, _C = 1, 2048, 1024, 2048, 4, 2, 2000
_FC = 512
_NFC = _F // _FC
_TT = 256
_NTT = _T // _TT
_EPAD = 128


def _pos_encoding(t, d):
    pos = np.arange(t)[:, None].astype(np.float32)
    i = np.arange(0, d, 2)[None, :].astype(np.float32)
    angle = pos / np.power(10000.0, i / d)
    pe = np.zeros((t, d), dtype=np.float32)
    pe[:, 0::2] = np.sin(angle)
    pe[:, 1::2] = np.cos(angle)
    return pe


_PE = _pos_encoding(_T, _D)


def _stage1_kernel(x_ref, fc1w_ref, fc1b_ref, g_ref, b_ref, mu_ref, var_ref,
                   pe_ref, rw_ref, rb_ref, h_ref, rlog_ref):
    h = jnp.dot(x_ref[0], fc1w_ref[...], preferred_element_type=jnp.float32)
    h = h + fc1b_ref[...]
    scale = g_ref[...] * jax.lax.rsqrt(var_ref[...] + 1e-5)
    shift = b_ref[...] - mu_ref[...] * scale
    h = h * scale + shift
    h = jnp.maximum(h, 0.0) + pe_ref[...]
    h_ref[...] = h
    rlog_ref[...] = jnp.dot(h, rw_ref[...], preferred_element_type=jnp.float32) + rb_ref[...]


def _stage1(x2, fc1_w, fc1_b, bn_gamma, bn_beta, bn_mean, bn_var, router_w, router_b):
    rw_p = jnp.zeros((_D, _EPAD), jnp.float32).at[:, :_E].set(router_w)
    rb_p = jnp.zeros((1, _EPAD), jnp.float32).at[0, :_E].set(router_b)
    row = lambda v: v.reshape(1, _D)
    pe = jnp.asarray(_PE)
    h, rlog = pl.pallas_call(
        _stage1_kernel,
        grid=(_NTT,),
        in_specs=[
            pl.BlockSpec((1, _TT, _D), lambda i: (0, i, 0)),
            pl.BlockSpec((_D, _D), lambda i: (0, 0)),
            pl.BlockSpec((1, _D), lambda i: (0, 0)),
            pl.BlockSpec((1, _D), lambda i: (0, 0)),
            pl.BlockSpec((1, _D), lambda i: (0, 0)),
            pl.BlockSpec((1, _D), lambda i: (0, 0)),
            pl.BlockSpec((1, _D), lambda i: (0, 0)),
            pl.BlockSpec((_TT, _D), lambda i: (i, 0)),
            pl.BlockSpec((_D, _EPAD), lambda i: (0, 0)),
            pl.BlockSpec((1, _EPAD), lambda i: (0, 0)),
        ],
        out_specs=[
            pl.BlockSpec((_TT, _D), lambda i: (i, 0)),
            pl.BlockSpec((_TT, _EPAD), lambda i: (i, 0)),
        ],
        out_shape=[
            jax.ShapeDtypeStruct((_T, _D), jnp.float32),
            jax.ShapeDtypeStruct((_T, _EPAD), jnp.float32),
        ],
    )(x2, fc1_w, row(fc1_b), row(bn_gamma), row(bn_beta), row(bn_mean),
      row(bn_var), pe, rw_p, rb_p)
    return h, rlog


def _route_kernel(rlog_ref, comb_ref, aux_ref):
    cols = [rlog_ref[:, i:i + 1] for i in range(_E)]
    m = jnp.maximum(jnp.maximum(cols[0], cols[1]), jnp.maximum(cols[2], cols[3]))
    es = [jnp.exp(c - m) for c in cols]
    ssum = es[0] + es[1] + es[2] + es[3]
    gates = [e / ssum for e in es]
    m1 = jnp.maximum(jnp.maximum(gates[0], gates[1]), jnp.maximum(gates[2], gates[3]))
    i1 = jnp.where(gates[0] == m1, 0,
         jnp.where(gates[1] == m1, 1,
         jnp.where(gates[2] == m1, 2, 3))).astype(jnp.int32)
    g2 = [jnp.where(i1 == i, -1.0, gates[i]) for i in range(_E)]
    m2 = jnp.maximum(jnp.maximum(g2[0], g2[1]), jnp.maximum(g2[2], g2[3]))
    i2 = jnp.where(g2[0] == m2, 0,
         jnp.where(g2[1] == m2, 1,
         jnp.where(g2[2] == m2, 2, 3))).astype(jnp.int32)
    denom = m1 + m2
    aux_acc = None
    for i in range(_E):
        sel = jnp.logical_or(i1 == i, i2 == i)
        ci = jnp.where(sel, gates[i] / denom, 0.0)
        comb_ref[:, i:i + 1] = ci
        imp = jnp.sum(gates[i], axis=0, keepdims=True) / _T
        load = jnp.sum((ci > 0).astype(jnp.float32), axis=0, keepdims=True) / _T
        term = imp * load
        aux_acc = term if aux_acc is None else aux_acc + term
    aux_ref[...] = _E * aux_acc


def _stage2(rlog):
    comb, aux = pl.pallas_call(
        _route_kernel,
        in_specs=[pl.BlockSpec((_T, _EPAD), lambda: (0, 0))],
        out_specs=[
            pl.BlockSpec((_T, _E), lambda: (0, 0)),
            pl.BlockSpec((1, 1), lambda: (0, 0)),
        ],
        out_shape=[
            jax.ShapeDtypeStruct((_T, _E), jnp.float32),
            jax.ShapeDtypeStruct((1, 1), jnp.float32),
        ],
    )(rlog)
    return comb, aux


def _moe_kernel(hpad_ref, w1_ref, b1_ref, w2_ref, b2_ref, comb_ref, out_ref, y1_ref):
    e = pl.program_id(0)
    fc = pl.program_id(1)

    @pl.when(jnp.logical_and(e == 0, fc == 0))
    def _init():
        out_ref[...] = jnp.zeros_like(out_ref)

    hp = hpad_ref[...]
    w1 = w1_ref[0].astype(jnp.bfloat16)
    b1 = b1_ref[0]
    w2 = w2_ref[0].astype(jnp.bfloat16)
    b2 = b2_ref[0]
    c = comb_ref[0]

    y1 = jnp.dot(hp[1:_T + 1], w1[0], preferred_element_type=jnp.float32)
    y1 += jnp.dot(hp[2:_T + 2], w1[1], preferred_element_type=jnp.float32)
    y1 += jnp.dot(hp[3:_T + 3], w1[2], preferred_element_type=jnp.float32)
    y1 = jnp.maximum(y1 + b1, 0.0).astype(jnp.bfloat16)
    y1_ref[0:1, :] = jnp.zeros((1, _FC), jnp.bfloat16)
    y1_ref[1:_T + 1, :] = y1
    y1_ref[_T + 1:_T + 2, :] = jnp.zeros((1, _FC), jnp.bfloat16)

    y = jnp.dot(y1_ref[0:_T], w2[0], preferred_element_type=jnp.float32)
    y += jnp.dot(y1_ref[1:_T + 1], w2[1], preferred_element_type=jnp.float32)
    y += jnp.dot(y1_ref[2:_T + 2], w2[2], preferred_element_type=jnp.float32)

    @pl.when(fc == 0)
    def _bias():
        out_ref[...] += c * b2

    out_ref[...] += c * y


def _stage3(h, exp_w1, exp_b1, exp_w2, exp_b2, comb):
    hpad = jnp.pad(h, ((2, 6), (0, 0))).astype(jnp.bfloat16)
    comb_t = comb.T.reshape(_E, _T, 1)
    b1r = exp_b1.reshape(_E, 1, _F)
    b2r = exp_b2.reshape(_E, 1, _D)
    moe = pl.pallas_call(
        _moe_kernel,
        grid=(_E, _NFC),
        in_specs=[
            pl.BlockSpec((_T + 8, _D), lambda e, fc: (0, 0)),
            pl.BlockSpec((1, 3, _D, _FC), lambda e, fc: (e, 0, 0, fc)),
            pl.BlockSpec((1, 1, _FC), lambda e, fc: (e, 0, fc)),
            pl.BlockSpec((1, 3, _FC, _D), lambda e, fc: (e, 0, fc, 0)),
            pl.BlockSpec((1, 1, _D), lambda e, fc: (e, 0, 0)),
            pl.BlockSpec((1, _T, 1), lambda e, fc: (e, 0, 0)),
        ],
        out_specs=pl.BlockSpec((_T, _D), lambda e, fc: (0, 0)),
        out_shape=jax.ShapeDtypeStruct((_T, _D), jnp.float32),
        scratch_shapes=[pltpu.VMEM((_T + 8, _FC), jnp.bfloat16)],
    )(hpad, exp_w1, b1r, exp_w2, b2r, comb_t)
    return moe


def _head_kernel(moe_ref, lng_ref, lnb_ref, ow_ref, ob_ref,
                 feat_ref, featn_ref, logits_ref, logp_ref, p_ref):
    m = moe_ref[...]
    mu = jnp.mean(m, axis=-1, keepdims=True)
    d = m - mu
    var = jnp.mean(d * d, axis=-1, keepdims=True)
    feat = d * jax.lax.rsqrt(var + 1e-6) * lng_ref[...] + lnb_ref[...]
    feat_ref[0] = feat
    n = jnp.sqrt(jnp.sum(feat * feat, axis=-1, keepdims=True))
    featn_ref[0] = feat / jnp.maximum(n, 1e-12)
    logits = jnp.dot(feat.astype(jnp.bfloat16), ow_ref[...].astype(jnp.bfloat16),
                     preferred_element_type=jnp.float32) + ob_ref[...]
    logits_ref[0] = logits
    mx = jnp.max(logits, axis=-1, keepdims=True)
    ex = jnp.exp(logits - mx)
    s = jnp.sum(ex, axis=-1, keepdims=True)
    logp_ref[0] = logits - (mx + jnp.log(s))
    p_ref[0] = ex / s


def _stage4(moe, ln_gamma, ln_beta, out_w, out_b):
    row = lambda v, n: v.reshape(1, n)
    outs = pl.pallas_call(
        _head_kernel,
        grid=(_NTT,),
        in_specs=[
            pl.BlockSpec((_TT, _D), lambda i: (i, 0)),
            pl.BlockSpec((1, _D), lambda i: (0, 0)),
            pl.BlockSpec((1, _D), lambda i: (0, 0)),
            pl.BlockSpec((_D, _C), lambda i: (0, 0)),
            pl.BlockSpec((1, _C), lambda i: (0, 0)),
        ],
        out_specs=[
            pl.BlockSpec((1, _TT, _D), lambda i: (0, i, 0)),
            pl.BlockSpec((1, _TT, _D), lambda i: (0, i, 0)),
            pl.BlockSpec((1, _TT, _C), lambda i: (0, i, 0)),
            pl.BlockSpec((1, _TT, _C), lambda i: (0, i, 0)),
            pl.BlockSpec((1, _TT, _C), lambda i: (0, i, 0)),
        ],
        out_shape=[
            jax.ShapeDtypeStruct((_B, _T, _D), jnp.float32),
            jax.ShapeDtypeStruct((_B, _T, _D), jnp.float32),
            jax.ShapeDtypeStruct((_B, _T, _C), jnp.float32),
            jax.ShapeDtypeStruct((_B, _T, _C), jnp.float32),
            jax.ShapeDtypeStruct((_B, _T, _C), jnp.float32),
        ],
    )(moe, row(ln_gamma, _D), row(ln_beta, _D), out_w, row(out_b, _C))
    return outs


def kernel(x, mask, fc1_w, fc1_b, bn_gamma, bn_beta, bn_mean, bn_var,
           router_w, router_b, exp_w1, exp_b1, exp_w2, exp_b2,
           ln_gamma, ln_beta, out_w, out_b):
    h, rlog = _stage1(x, fc1_w, fc1_b, bn_gamma, bn_beta, bn_mean, bn_var,
                      router_w, router_b)
    comb, aux = _stage2(rlog)
    moe = _stage3(h, exp_w1, exp_b1, exp_w2, exp_b2, comb)
    feat, featn, logits, logp, p = _stage4(moe, ln_gamma, ln_beta, out_w, out_b)
    return (feat, featn, logits, logp, p, aux[0, 0])


# speedup vs baseline: 1.6012x; 1.0374x over previous
import numpy as np
import jax
import jax.numpy as jnp
from jax.experimental import pallas as pl
from jax.experimental.pallas import tpu as pltpu

_B, _T, _D, _F, _E, _K, _C = 1, 2048, 1024, 2048, 4, 2, 2000
_FC = 512
_NFC = _F // _FC
_TT = 256
_NTT = _T // _TT
_EPAD = 128


def _pos_encoding(t, d):
    pos = np.arange(t)[:, None].astype(np.float32)
    i = np.arange(0, d, 2)[None, :].astype(np.float32)
    angle = pos / np.power(10000.0, i / d)
    pe = np.zeros((t, d), dtype=np.float32)
    pe[:, 0::2] = np.sin(angle)
    pe[:, 1::2] = np.cos(angle)
    return pe


_PE = _pos_encoding(_T, _D)


def _stage1_kernel(x_ref, fc1w_ref, fc1b_ref, g_ref, b_ref, mu_ref, var_ref,
                   pe_ref, rw_ref, rb_ref, h_ref, rlog_ref):
    h = jnp.dot(x_ref[0], fc1w_ref[...], preferred_element_type=jnp.float32)
    h = h + fc1b_ref[...]
    scale = g_ref[...] * jax.lax.rsqrt(var_ref[...] + 1e-5)
    shift = b_ref[...] - mu_ref[...] * scale
    h = h * scale + shift
    h = jnp.maximum(h, 0.0) + pe_ref[...]
    h_ref[...] = h
    rlog_ref[...] = jnp.dot(h, rw_ref[...], preferred_element_type=jnp.float32) + rb_ref[...]


def _stage1(x2, fc1_w, fc1_b, bn_gamma, bn_beta, bn_mean, bn_var, router_w, router_b):
    rw_p = jnp.zeros((_D, _EPAD), jnp.float32).at[:, :_E].set(router_w)
    rb_p = jnp.zeros((1, _EPAD), jnp.float32).at[0, :_E].set(router_b)
    row = lambda v: v.reshape(1, _D)
    pe = jnp.asarray(_PE)
    h, rlog = pl.pallas_call(
        _stage1_kernel,
        grid=(_NTT,),
        in_specs=[
            pl.BlockSpec((1, _TT, _D), lambda i: (0, i, 0)),
            pl.BlockSpec((_D, _D), lambda i: (0, 0)),
            pl.BlockSpec((1, _D), lambda i: (0, 0)),
            pl.BlockSpec((1, _D), lambda i: (0, 0)),
            pl.BlockSpec((1, _D), lambda i: (0, 0)),
            pl.BlockSpec((1, _D), lambda i: (0, 0)),
            pl.BlockSpec((1, _D), lambda i: (0, 0)),
            pl.BlockSpec((_TT, _D), lambda i: (i, 0)),
            pl.BlockSpec((_D, _EPAD), lambda i: (0, 0)),
            pl.BlockSpec((1, _EPAD), lambda i: (0, 0)),
        ],
        out_specs=[
            pl.BlockSpec((_TT, _D), lambda i: (i, 0)),
            pl.BlockSpec((_TT, _EPAD), lambda i: (i, 0)),
        ],
        out_shape=[
            jax.ShapeDtypeStruct((_T, _D), jnp.float32),
            jax.ShapeDtypeStruct((_T, _EPAD), jnp.float32),
        ],
    )(x2, fc1_w, row(fc1_b), row(bn_gamma), row(bn_beta), row(bn_mean),
      row(bn_var), pe, rw_p, rb_p)
    return h, rlog


def _route_kernel(rlog_ref, comb_ref, aux_ref):
    cols = [rlog_ref[:, i:i + 1] for i in range(_E)]
    m = jnp.maximum(jnp.maximum(cols[0], cols[1]), jnp.maximum(cols[2], cols[3]))
    es = [jnp.exp(c - m) for c in cols]
    ssum = es[0] + es[1] + es[2] + es[3]
    gates = [e / ssum for e in es]
    m1 = jnp.maximum(jnp.maximum(gates[0], gates[1]), jnp.maximum(gates[2], gates[3]))
    i1 = jnp.where(gates[0] == m1, 0,
         jnp.where(gates[1] == m1, 1,
         jnp.where(gates[2] == m1, 2, 3))).astype(jnp.int32)
    g2 = [jnp.where(i1 == i, -1.0, gates[i]) for i in range(_E)]
    m2 = jnp.maximum(jnp.maximum(g2[0], g2[1]), jnp.maximum(g2[2], g2[3]))
    i2 = jnp.where(g2[0] == m2, 0,
         jnp.where(g2[1] == m2, 1,
         jnp.where(g2[2] == m2, 2, 3))).astype(jnp.int32)
    denom = m1 + m2
    aux_acc = None
    for i in range(_E):
        sel = jnp.logical_or(i1 == i, i2 == i)
        ci = jnp.where(sel, gates[i] / denom, 0.0)
        comb_ref[:, i:i + 1] = ci
        imp = jnp.sum(gates[i], axis=0, keepdims=True) / _T
        load = jnp.sum((ci > 0).astype(jnp.float32), axis=0, keepdims=True) / _T
        term = imp * load
        aux_acc = term if aux_acc is None else aux_acc + term
    aux_ref[...] = _E * aux_acc


def _stage2(rlog):
    comb, aux = pl.pallas_call(
        _route_kernel,
        in_specs=[pl.BlockSpec((_T, _EPAD), lambda: (0, 0))],
        out_specs=[
            pl.BlockSpec((_T, _E), lambda: (0, 0)),
            pl.BlockSpec((1, 1), lambda: (0, 0)),
        ],
        out_shape=[
            jax.ShapeDtypeStruct((_T, _E), jnp.float32),
            jax.ShapeDtypeStruct((1, 1), jnp.float32),
        ],
    )(rlog)
    return comb, aux


def _moe_kernel(hp3_ref, w1_ref, b1_ref, w2_ref, b2_ref, comb_ref, out_ref, y1_ref):
    e = pl.program_id(0)
    fc = pl.program_id(1)

    @pl.when(jnp.logical_and(e == 0, fc == 0))
    def _init():
        out_ref[...] = jnp.zeros_like(out_ref)

    w1 = w1_ref[0].reshape(3 * _D, _FC).astype(jnp.bfloat16)
    b1 = b1_ref[0]
    w2 = w2_ref[0].astype(jnp.bfloat16)
    b2 = b2_ref[0]
    lane = jax.lax.broadcasted_iota(jnp.int32, (_T, _E), 1)
    c = jnp.sum(jnp.where(lane == e, comb_ref[...], 0.0), axis=1, keepdims=True)

    y1 = jnp.dot(hp3_ref[...], w1, preferred_element_type=jnp.float32)
    y1 = jnp.maximum(y1 + b1, 0.0).astype(jnp.bfloat16)
    y1_ref[0:1, :] = jnp.zeros((1, _FC), jnp.bfloat16)
    y1_ref[1:_T + 1, :] = y1
    y1_ref[_T + 1:_T + 2, :] = jnp.zeros((1, _FC), jnp.bfloat16)

    y = jnp.dot(y1_ref[0:_T], w2[0], preferred_element_type=jnp.float32)
    y += jnp.dot(y1_ref[1:_T + 1], w2[1], preferred_element_type=jnp.float32)
    y += jnp.dot(y1_ref[2:_T + 2], w2[2], preferred_element_type=jnp.float32)

    @pl.when(fc == 0)
    def _bias():
        out_ref[...] += c * b2

    out_ref[...] += c * y


def _stage3(h, exp_w1, exp_b1, exp_w2, exp_b2, comb):
    hpad = jnp.pad(h, ((1, 1), (0, 0)))
    hp3 = jnp.concatenate(
        [hpad[0:_T], hpad[1:_T + 1], hpad[2:_T + 2]], axis=1
    ).astype(jnp.bfloat16)
    b1r = exp_b1.reshape(_E, 1, _F)
    b2r = exp_b2.reshape(_E, 1, _D)
    moe = pl.pallas_call(
        _moe_kernel,
        grid=(_E, _NFC),
        in_specs=[
            pl.BlockSpec((_T, 3 * _D), lambda e, fc: (0, 0)),
            pl.BlockSpec((1, 3, _D, _FC), lambda e, fc: (e, 0, 0, fc)),
            pl.BlockSpec((1, 1, _FC), lambda e, fc: (e, 0, fc)),
            pl.BlockSpec((1, 3, _FC, _D), lambda e, fc: (e, 0, fc, 0)),
            pl.BlockSpec((1, 1, _D), lambda e, fc: (e, 0, 0)),
            pl.BlockSpec((_T, _E), lambda e, fc: (0, 0)),
        ],
        out_specs=pl.BlockSpec((_T, _D), lambda e, fc: (0, 0)),
        out_shape=jax.ShapeDtypeStruct((_T, _D), jnp.float32),
        scratch_shapes=[pltpu.VMEM((_T + 8, _FC), jnp.bfloat16)],
    )(hp3, exp_w1, b1r, exp_w2, b2r, comb)
    return moe


def _head_kernel(moe_ref, lng_ref, lnb_ref, ow_ref, ob_ref,
                 feat_ref, featn_ref, logits_ref, logp_ref, p_ref):
    m = moe_ref[...]
    mu = jnp.mean(m, axis=-1, keepdims=True)
    d = m - mu
    var = jnp.mean(d * d, axis=-1, keepdims=True)
    feat = d * jax.lax.rsqrt(var + 1e-6) * lng_ref[...] + lnb_ref[...]
    feat_ref[0] = feat
    n = jnp.sqrt(jnp.sum(feat * feat, axis=-1, keepdims=True))
    featn_ref[0] = feat / jnp.maximum(n, 1e-12)
    logits = jnp.dot(feat.astype(jnp.bfloat16), ow_ref[...].astype(jnp.bfloat16),
                     preferred_element_type=jnp.float32) + ob_ref[...]
    logits_ref[0] = logits
    mx = jnp.max(logits, axis=-1, keepdims=True)
    ex = jnp.exp(logits - mx)
    s = jnp.sum(ex, axis=-1, keepdims=True)
    logp_ref[0] = logits - (mx + jnp.log(s))
    p_ref[0] = ex / s


def _stage4(moe, ln_gamma, ln_beta, out_w, out_b):
    row = lambda v, n: v.reshape(1, n)
    outs = pl.pallas_call(
        _head_kernel,
        grid=(_NTT,),
        in_specs=[
            pl.BlockSpec((_TT, _D), lambda i: (i, 0)),
            pl.BlockSpec((1, _D), lambda i: (0, 0)),
            pl.BlockSpec((1, _D), lambda i: (0, 0)),
            pl.BlockSpec((_D, _C), lambda i: (0, 0)),
            pl.BlockSpec((1, _C), lambda i: (0, 0)),
        ],
        out_specs=[
            pl.BlockSpec((1, _TT, _D), lambda i: (0, i, 0)),
            pl.BlockSpec((1, _TT, _D), lambda i: (0, i, 0)),
            pl.BlockSpec((1, _TT, _C), lambda i: (0, i, 0)),
            pl.BlockSpec((1, _TT, _C), lambda i: (0, i, 0)),
            pl.BlockSpec((1, _TT, _C), lambda i: (0, i, 0)),
        ],
        out_shape=[
            jax.ShapeDtypeStruct((_B, _T, _D), jnp.float32),
            jax.ShapeDtypeStruct((_B, _T, _D), jnp.float32),
            jax.ShapeDtypeStruct((_B, _T, _C), jnp.float32),
            jax.ShapeDtypeStruct((_B, _T, _C), jnp.float32),
            jax.ShapeDtypeStruct((_B, _T, _C), jnp.float32),
        ],
    )(moe, row(ln_gamma, _D), row(ln_beta, _D), out_w, row(out_b, _C))
    return outs


def kernel(x, mask, fc1_w, fc1_b, bn_gamma, bn_beta, bn_mean, bn_var,
           router_w, router_b, exp_w1, exp_b1, exp_w2, exp_b2,
           ln_gamma, ln_beta, out_w, out_b):
    h, rlog = _stage1(x, fc1_w, fc1_b, bn_gamma, bn_beta, bn_mean, bn_var,
                      router_w, router_b)
    comb, aux = _stage2(rlog)
    moe = _stage3(h, exp_w1, exp_b1, exp_w2, exp_b2, comb)
    feat, featn, logits, logp, p = _stage4(moe, ln_gamma, ln_beta, out_w, out_b)
    return (feat, featn, logits, logp, p, aux[0, 0])
